# baseline (speedup 1.0000x reference)
import dataclasses
import math

import jax
import jax.numpy as jnp
from jax import lax
from jax.experimental import pallas as pl
from jax.experimental.pallas import tpu as pltpu
from jax.experimental.pallas import tpu_sc as plsc

N_CH = 128
IN_BS = 131072
OUT_BS = math.ceil(IN_BS * 48000 / 44100)

NUM_WORKERS = 32
CW = 4464
OUT_PAD = NUM_WORKERS * CW
WB = 4160


def _resample_kernel(x_hbm, fr_hbm, fl_hbm, cl_hbm, out_hbm,
                     fl_v, cl_v, fr_v, win_v, ob_v):
    wid = lax.axis_index("s") * 2 + lax.axis_index("c")
    c0 = pl.multiple_of(wid * CW, 8)

    pltpu.sync_copy(fl_hbm.at[pl.ds(c0, CW)], fl_v)
    pltpu.sync_copy(cl_hbm.at[pl.ds(c0, CW)], cl_v)
    pltpu.sync_copy(fr_hbm.at[pl.ds(c0, CW)], fr_v)

    floor0 = fl_v[pl.ds(0, 16)][0]
    start8 = pl.multiple_of(
        jnp.minimum(jnp.bitwise_and(floor0, -8), IN_BS - WB), 8)

    @pl.loop(0, CW, step=16)
    def _(j):
        fl_v[pl.ds(j, 16)] = fl_v[pl.ds(j, 16)] - start8
        cl_v[pl.ds(j, 16)] = cl_v[pl.ds(j, 16)] - start8

    @pl.loop(0, N_CH)
    def _(ch):
        off_in = pl.multiple_of(ch * IN_BS + start8, 8)
        pltpu.sync_copy(x_hbm.at[pl.ds(off_in, WB)], win_v)

        @pl.loop(0, CW, step=16)
        def _(j):
            sl = pl.ds(j, 16)
            a = plsc.load_gather(win_v, [fl_v[sl]])
            b = plsc.load_gather(win_v, [cl_v[sl]])
            ob_v[sl] = a + (b - a) * fr_v[sl]

        off_out = pl.multiple_of(ch * OUT_PAD + c0, 8)
        pltpu.sync_copy(ob_v, out_hbm.at[pl.ds(off_out, CW)])


def kernel(x, interp_in, floor_in, ceil_in):
    pad = OUT_PAD - OUT_BS
    fr = jnp.pad(interp_in, (0, pad))
    fl = jnp.pad(floor_in.astype(jnp.int32), (0, pad), constant_values=IN_BS - 1)
    cl = jnp.pad(ceil_in.astype(jnp.int32), (0, pad), constant_values=IN_BS - 1)
    x_flat = x.reshape(-1)

    cp = pltpu.CompilerParams()
    if "needs_layout_passes" in pltpu.CompilerParams.__dataclass_fields__:
        cp = dataclasses.replace(cp, needs_layout_passes=False)
    mesh = plsc.VectorSubcoreMesh(core_axis_name="c", subcore_axis_name="s")
    run = pl.kernel(
        _resample_kernel,
        out_type=jax.ShapeDtypeStruct((N_CH * OUT_PAD,), jnp.float32),
        mesh=mesh,
        compiler_params=cp,
        scratch_types=[
            pltpu.VMEM((CW,), jnp.int32),
            pltpu.VMEM((CW,), jnp.int32),
            pltpu.VMEM((CW,), jnp.float32),
            pltpu.VMEM((WB,), jnp.float32),
            pltpu.VMEM((CW,), jnp.float32),
        ],
    )
    out = run(x_flat, fr, fl, cl)
    return out.reshape(N_CH, OUT_PAD)[:, :OUT_BS]


# speedup vs baseline: 1.2474x; 1.2474x over previous
import dataclasses
import math

import jax
import jax.numpy as jnp
from jax import lax
from jax.experimental import pallas as pl
from jax.experimental.pallas import tpu as pltpu
from jax.experimental.pallas import tpu_sc as plsc

N_CH = 128
IN_BS = 131072
OUT_BS = math.ceil(IN_BS * 48000 / 44100)

NUM_WORKERS = 32
CW = 4464
OUT_PAD = NUM_WORKERS * CW
WB = 4160
K = 4
NT = N_CH // K
NBUF = 2


def _resample_kernel(x_hbm, fr_hbm, fl_hbm, cl_hbm, out_hbm,
                     fl_v, cl_v, fr_v, *rest):
    win = [[rest[b * K + k] for k in range(K)] for b in range(NBUF)]
    ob = [[rest[NBUF * K + b * K + k] for k in range(K)] for b in range(NBUF)]
    sem_in = rest[2 * NBUF * K:2 * NBUF * K + NBUF]
    sem_out = rest[2 * NBUF * K + NBUF:]
    wid = lax.axis_index("s") * 2 + lax.axis_index("c")
    c0 = pl.multiple_of(wid * CW, 8)

    pltpu.sync_copy(fl_hbm.at[pl.ds(c0, CW)], fl_v)
    pltpu.sync_copy(cl_hbm.at[pl.ds(c0, CW)], cl_v)
    pltpu.sync_copy(fr_hbm.at[pl.ds(c0, CW)], fr_v)

    floor0 = fl_v[pl.ds(0, 16)][0]
    start8 = pl.multiple_of(
        jnp.minimum(jnp.bitwise_and(floor0, -8), IN_BS - WB), 8)

    @pl.loop(0, CW, step=16)
    def _(j):
        fl_v[pl.ds(j, 16)] = fl_v[pl.ds(j, 16)] - start8
        cl_v[pl.ds(j, 16)] = cl_v[pl.ds(j, 16)] - start8

    def fire_in(t, b):
        for k in range(K):
            off = pl.multiple_of((t * K + k) * IN_BS + start8, 8)
            pltpu.async_copy(
                x_hbm.at[pl.ds(off, WB)], win[b][k], sem_in[b])

    def wait_in(b):
        for k in range(K):
            pltpu.make_async_copy(
                x_hbm.at[pl.ds(0, WB)], win[b][k], sem_in[b]).wait()

    def fire_out(t, b):
        for k in range(K):
            off = pl.multiple_of((t * K + k) * OUT_PAD + c0, 8)
            pltpu.async_copy(
                ob[b][k], out_hbm.at[pl.ds(off, CW)], sem_out[b])

    def wait_out(b):
        for k in range(K):
            pltpu.make_async_copy(
                ob[b][k], out_hbm.at[pl.ds(0, CW)], sem_out[b]).wait()

    def compute(b):
        @pl.loop(0, CW, step=16)
        def _(j):
            sl = pl.ds(j, 16)
            lv = fl_v[sl]
            l2 = cl_v[sl]
            f = fr_v[sl]
            for k in range(K):
                a = plsc.load_gather(win[b][k], [lv])
                bb = plsc.load_gather(win[b][k], [l2])
                ob[b][k][sl] = a + (bb - a) * f

    fire_in(0, 0)

    @pl.loop(0, NT, step=NBUF)
    def _(t):
        for b in range(NBUF):
            tt = t + b

            @pl.when(tt + 1 < NT)
            def _():
                fire_in(tt + 1, (b + 1) % NBUF)

            wait_in(b)

            @pl.when(tt >= NBUF)
            def _():
                wait_out(b)

            compute(b)
            fire_out(tt, b)

    for b in range(NBUF):
        wait_out(b)


def kernel(x, interp_in, floor_in, ceil_in):
    pad = OUT_PAD - OUT_BS
    fr = jnp.pad(interp_in, (0, pad))
    fl = jnp.pad(floor_in.astype(jnp.int32), (0, pad), constant_values=IN_BS - 1)
    cl = jnp.pad(ceil_in.astype(jnp.int32), (0, pad), constant_values=IN_BS - 1)

    cp = pltpu.CompilerParams()
    if "needs_layout_passes" in pltpu.CompilerParams.__dataclass_fields__:
        cp = dataclasses.replace(cp, needs_layout_passes=False)
    mesh = plsc.VectorSubcoreMesh(core_axis_name="c", subcore_axis_name="s")
    run = pl.kernel(
        _resample_kernel,
        out_type=jax.ShapeDtypeStruct((N_CH * OUT_PAD,), jnp.float32),
        mesh=mesh,
        compiler_params=cp,
        scratch_types=[
            pltpu.VMEM((CW,), jnp.int32),
            pltpu.VMEM((CW,), jnp.int32),
            pltpu.VMEM((CW,), jnp.float32),
            *[pltpu.VMEM((WB,), jnp.float32) for _ in range(NBUF * K)],
            *[pltpu.VMEM((CW,), jnp.float32) for _ in range(NBUF * K)],
            *[pltpu.SemaphoreType.DMA for _ in range(2 * NBUF)],
        ],
    )
    out = run(x.reshape(-1), fr, fl, cl)
    return out.reshape(N_CH, OUT_PAD)[:, :OUT_BS]


# speedup vs baseline: 2.1274x; 1.7055x over previous
import dataclasses
import math

import jax
import jax.numpy as jnp
from jax import lax
from jax.experimental import pallas as pl
from jax.experimental.pallas import tpu as pltpu
from jax.experimental.pallas import tpu_sc as plsc

N_CH = 128
IN_BS = 131072
OUT_BS = math.ceil(IN_BS * 48000 / 44100)

NUM_WORKERS = 32
CW = 4464
OUT_PAD = NUM_WORKERS * CW
WB = 4160
K = 4
NT = N_CH // K
NBUF = 2


def _resample_kernel(x_hbm, fr_hbm, fl_hbm, out_hbm,
                     fl_v, fr_v, *rest):
    win = [[rest[b * K + k] for k in range(K)] for b in range(NBUF)]
    ob = [[rest[NBUF * K + b * K + k] for k in range(K)] for b in range(NBUF)]
    sem_in = rest[2 * NBUF * K:2 * NBUF * K + NBUF]
    sem_out = rest[2 * NBUF * K + NBUF:]
    wid = lax.axis_index("s") * 2 + lax.axis_index("c")
    c0 = pl.multiple_of(wid * CW, 8)

    pltpu.sync_copy(fl_hbm.at[pl.ds(c0, CW)], fl_v)
    pltpu.sync_copy(fr_hbm.at[pl.ds(c0, CW)], fr_v)

    floor0 = fl_v[pl.ds(0, 16)][0]
    start8 = pl.multiple_of(
        jnp.minimum(jnp.bitwise_and(floor0, -8), IN_BS - WB), 8)

    @plsc.parallel_loop(0, CW, step=16, unroll=4)
    def _(j):
        fl_v[pl.ds(j, 16)] = fl_v[pl.ds(j, 16)] - start8

    def fire_in(t, b):
        for k in range(K):
            off = pl.multiple_of((t * K + k) * IN_BS + start8, 8)
            pltpu.async_copy(
                x_hbm.at[pl.ds(off, WB)], win[b][k], sem_in[b])

    def wait_in(b):
        for k in range(K):
            pltpu.make_async_copy(
                x_hbm.at[pl.ds(0, WB)], win[b][k], sem_in[b]).wait()

    def fire_out(t, b):
        for k in range(K):
            off = pl.multiple_of((t * K + k) * OUT_PAD + c0, 8)
            pltpu.async_copy(
                ob[b][k], out_hbm.at[pl.ds(off, CW)], sem_out[b])

    def wait_out(b):
        for k in range(K):
            pltpu.make_async_copy(
                ob[b][k], out_hbm.at[pl.ds(0, CW)], sem_out[b]).wait()

    def compute(b):
        @plsc.parallel_loop(0, CW, step=16, unroll=4)
        def _(j):
            sl = pl.ds(j, 16)
            lv = fl_v[sl]
            l2 = jnp.minimum(lv + 1, WB - 1)
            f = fr_v[sl]
            for k in range(K):
                a = plsc.load_gather(win[b][k], [lv])
                bb = plsc.load_gather(win[b][k], [l2])
                ob[b][k][sl] = a + (bb - a) * f

    fire_in(0, 0)

    @pl.loop(0, NT, step=NBUF)
    def _(t):
        for b in range(NBUF):
            tt = t + b

            @pl.when(tt + 1 < NT)
            def _():
                fire_in(tt + 1, (b + 1) % NBUF)

            wait_in(b)

            @pl.when(tt >= NBUF)
            def _():
                wait_out(b)

            compute(b)
            fire_out(tt, b)

    for b in range(NBUF):
        wait_out(b)


def kernel(x, interp_in, floor_in, ceil_in):
    pad = OUT_PAD - OUT_BS
    fr = jnp.pad(interp_in, (0, pad))
    fl = jnp.pad(floor_in.astype(jnp.int32), (0, pad), constant_values=IN_BS - 1)

    cp = pltpu.CompilerParams()
    if "needs_layout_passes" in pltpu.CompilerParams.__dataclass_fields__:
        cp = dataclasses.replace(cp, needs_layout_passes=False)
    mesh = plsc.VectorSubcoreMesh(core_axis_name="c", subcore_axis_name="s")
    run = pl.kernel(
        _resample_kernel,
        out_type=jax.ShapeDtypeStruct((N_CH * OUT_PAD,), jnp.float32),
        mesh=mesh,
        compiler_params=cp,
        scratch_types=[
            pltpu.VMEM((CW,), jnp.int32),
            pltpu.VMEM((CW,), jnp.float32),
            *[pltpu.VMEM((WB,), jnp.float32) for _ in range(NBUF * K)],
            *[pltpu.VMEM((CW,), jnp.float32) for _ in range(NBUF * K)],
            *[pltpu.SemaphoreType.DMA for _ in range(2 * NBUF)],
        ],
    )
    out = run(x.reshape(-1), fr, fl)
    return out.reshape(N_CH, OUT_PAD)[:, :OUT_BS]
